# baseline (speedup 1.0000x reference)
import jax
import jax.numpy as jnp
from jax import lax
from jax.experimental import pallas as pl
from jax.experimental.pallas import tpu as pltpu
from jax.experimental.pallas import tpu_sc as plsc

B = 16384
D = 64
NROWS = 1000000
DP = 2 * D
NPACK = NROWS // 2

_info = plsc.get_sparse_core_info()
NC = _info.num_cores
NS = _info.num_subcores
NW = NC * NS
B_PER_W = B // NW
CHUNK = 128
NCHUNK = B_PER_W // CHUNK


def _gather_body(idx_hbm, table_hbm, out_hbm,
                 idx_v, tidx_v, tbuf0, tbuf1, obuf0, obuf1,
                 gsem0, gsem1, osem0, osem1):
    wid = lax.axis_index("s") * NC + lax.axis_index("c")
    base = wid * B_PER_W
    pltpu.sync_copy(idx_hbm.at[pl.ds(base, B_PER_W)], idx_v)

    def tbody(g, carry):
        v = idx_v[pl.ds(g * 16, 16)]
        tidx_v[pl.ds(g * 16, 16)] = lax.shift_right_logical(v, 1)
        return carry

    lax.fori_loop(0, B_PER_W // 16, tbody, 0)

    tbufs = (tbuf0, tbuf1)
    obufs = (obuf0, obuf1)
    gsems = (gsem0, gsem1)
    osems = (osem0, osem1)

    def fire_gather(c, p):
        pltpu.make_async_copy(
            table_hbm.at[tidx_v.at[pl.ds(c * CHUNK, CHUNK)]],
            tbufs[p], gsems[p],
        ).start()

    fire_gather(0, 0)
    fire_gather(1, 1)

    def outer(k, carry):
        for p in range(2):
            c = 2 * k + p
            cstart = c * CHUNK
            pltpu.make_async_copy(
                table_hbm.at[tidx_v.at[pl.ds(cstart, CHUNK)]],
                tbufs[p], gsems[p],
            ).wait()
            @pl.when(k > 0)
            def _():
                pltpu.make_async_copy(
                    obufs[p], out_hbm.at[pl.ds(base, CHUNK)], osems[p],
                ).wait()
            for g in range(CHUNK // 16):
                vec = idx_v[pl.ds(cstart + g * 16, 16)]
                for j in range(16):
                    r = g * 16 + j
                    half = lax.mul(lax.bitwise_and(vec[j], 1), D)
                    for cc in range(D // 16):
                        obufs[p][r, pl.ds(cc * 16, 16)] = (
                            tbufs[p][r, pl.ds(half + cc * 16, 16)]
                        )
            pltpu.make_async_copy(
                obufs[p], out_hbm.at[pl.ds(base + cstart, CHUNK)], osems[p],
            ).start()

            @pl.when(k < NCHUNK // 2 - 1)
            def _():
                nstart = cstart + 2 * CHUNK
                pltpu.make_async_copy(
                    table_hbm.at[tidx_v.at[pl.ds(nstart, CHUNK)]],
                    tbufs[p], gsems[p],
                ).start()
        return carry

    lax.fori_loop(0, NCHUNK // 2, outer, 0)
    for p in range(2):
        pltpu.make_async_copy(
            obufs[p], out_hbm.at[pl.ds(base, CHUNK)], osems[p],
        ).wait()


@jax.jit
def kernel(batchInds, physiologicalProfile):
    packed = jnp.concatenate(
        [physiologicalProfile[0::2], physiologicalProfile[1::2]], axis=1
    )
    mesh = plsc.VectorSubcoreMesh(core_axis_name="c", subcore_axis_name="s")
    gather = pl.kernel(
        _gather_body,
        out_type=jax.ShapeDtypeStruct((B, D), jnp.float32),
        mesh=mesh,
        scratch_types=[
            pltpu.VMEM((B_PER_W,), jnp.int32),
            pltpu.VMEM((B_PER_W,), jnp.int32),
            pltpu.VMEM((CHUNK, DP), jnp.float32),
            pltpu.VMEM((CHUNK, DP), jnp.float32),
            pltpu.VMEM((CHUNK, D), jnp.float32),
            pltpu.VMEM((CHUNK, D), jnp.float32),
            pltpu.SemaphoreType.DMA,
            pltpu.SemaphoreType.DMA,
            pltpu.SemaphoreType.DMA,
            pltpu.SemaphoreType.DMA,
        ],
    )
    return gather(batchInds, packed)


# speedup vs baseline: 24.1776x; 24.1776x over previous
import jax
import jax.numpy as jnp
from jax import lax
from jax.experimental import pallas as pl
from jax.experimental.pallas import tpu as pltpu
from jax.experimental.pallas import tpu_sc as plsc

B = 16384
D = 64

_info = plsc.get_sparse_core_info()
NC = _info.num_cores
NS = _info.num_subcores
NW = NC * NS
B_PER_W = B // NW
NSEM = 8
ROWS_PER_SEM = B_PER_W // NSEM


def _gather_body(idx_hbm, table_hbm, out_hbm, idx_v, rows_v, *sems):
    wid = lax.axis_index("s") * NC + lax.axis_index("c")
    base = wid * B_PER_W
    pltpu.sync_copy(idx_hbm.at[pl.ds(base, B_PER_W)], idx_v)

    def body(c, carry):
        vec = idx_v[pl.ds(c * 16, 16)]
        for j in range(16):
            r = vec[j]
            pltpu.make_async_copy(
                table_hbm.at[r], rows_v.at[c * 16 + j], sems[j % NSEM]
            ).start()
        return carry

    lax.fori_loop(0, B_PER_W // 16, body, 0)
    for s in range(NSEM):
        pltpu.make_async_copy(
            table_hbm.at[pl.ds(0, ROWS_PER_SEM)],
            rows_v.at[pl.ds(0, ROWS_PER_SEM)],
            sems[s],
        ).wait()
    pltpu.sync_copy(rows_v, out_hbm.at[pl.ds(base, B_PER_W)])


@jax.jit
def kernel(batchInds, physiologicalProfile):
    mesh = plsc.VectorSubcoreMesh(core_axis_name="c", subcore_axis_name="s")
    k = pl.kernel(
        _gather_body,
        out_type=jax.ShapeDtypeStruct((B, D), jnp.float32),
        mesh=mesh,
        scratch_types=[
            pltpu.VMEM((B_PER_W,), jnp.int32),
            pltpu.VMEM((B_PER_W, D), jnp.float32),
        ] + [pltpu.SemaphoreType.DMA] * NSEM,
    )
    return k(batchInds, physiologicalProfile)
